# baseline (speedup 1.0000x reference)
import functools

import jax
import jax.numpy as jnp
from jax import lax
from jax.experimental import pallas as pl
from jax.experimental.pallas import tpu as pltpu
from jax.experimental.pallas import tpu_sc as plsc

_VOCAB = 1024
_SEQ_BLK = 512


def _fill_body(out_ref):
    j = pl.program_id(1)
    seq_len = pl.num_programs(1) * out_ref.shape[1]
    s = out_ref.shape[1]

    col = jax.lax.broadcasted_iota(jnp.int32, (s, _VOCAB), 1)
    pos = j * s + jax.lax.broadcasted_iota(jnp.int32, (s, _VOCAB), 0)

    base = jnp.where(
        col == 0, 1.0, jnp.where(col == 1, 0.5, jnp.where(col == 2, 0.0, -1000.0))
    )
    last_row = jnp.where(col == 0, 0.0, -1000.0)
    out_ref[0] = jnp.where(pos == seq_len - 1, last_row, base)


def _make_scatter(bsz, seq_len):
    n_pos = bsz * seq_len
    info = plsc.get_sparse_core_info()
    n_workers = info.num_cores * info.num_subcores
    per_w = n_pos // n_workers
    n_rows = per_w // 128
    mesh = plsc.VectorSubcoreMesh(core_axis_name="c", subcore_axis_name="s")

    @functools.partial(
        pl.kernel,
        mesh=mesh,
        scratch_types=[
            pltpu.VMEM((per_w,), jnp.int32),
            pltpu.VMEM((n_rows, 128), jnp.int32),
            pltpu.VMEM((n_rows, 128), jnp.float32),
            pltpu.SemaphoreType.DMA,
        ],
    )
    def scatter(tgt_hbm, out_ref, tgt_v, idx_v, val_v, sem):
        wid = lax.axis_index("s") * info.num_cores + lax.axis_index("c")
        q0 = wid * per_w
        pltpu.sync_copy(tgt_hbm.at[pl.ds(q0, per_w)], tgt_v)
        for r in range(n_rows):
            for c in range(8):
                j = r * 8 + c
                qv = q0 + j * 16 + lax.iota(jnp.int32, 16)
                tgt = tgt_v[pl.ds(j * 16, 16)]
                is_last = (qv & (seq_len - 1)) == (seq_len - 1)
                idx = qv * _VOCAB + jnp.where(is_last, _VOCAB - 1, tgt)
                idx_v[r, pl.ds(c * 16, 16)] = idx
                val_v[r, pl.ds(c * 16, 16)] = jnp.where(is_last, -1000.0, 5.0)
        copies = [
            pltpu.async_copy(val_v.at[r], out_ref.at[idx_v.at[r]], sem)
            for r in range(n_rows)
        ]
        for cp in copies:
            cp.wait()

    return scatter


def kernel(input_ids):
    bsz, seq_len = input_ids.shape
    loss = jnp.asarray(0.25, dtype=jnp.float32)
    n_blk = seq_len // _SEQ_BLK

    background = pl.pallas_call(
        _fill_body,
        grid=(bsz, n_blk),
        out_specs=pl.BlockSpec((1, _SEQ_BLK, _VOCAB), lambda b, j: (b, j, 0)),
        out_shape=jax.ShapeDtypeStruct((bsz, seq_len, _VOCAB), jnp.float32),
    )()

    targets = jnp.concatenate(
        [input_ids[:, 1:] % _VOCAB, jnp.zeros((bsz, 1), jnp.int32)], axis=1
    ).reshape(-1)

    out_ref = jax.new_ref(background.reshape(-1))
    _make_scatter(bsz, seq_len)(targets, out_ref)
    logits = out_ref[...].reshape(bsz, seq_len, _VOCAB)
    return loss, logits


# speedup vs baseline: 1.0030x; 1.0030x over previous
import functools

import jax
import jax.numpy as jnp
from jax import lax
from jax.experimental import pallas as pl
from jax.experimental.pallas import tpu as pltpu
from jax.experimental.pallas import tpu_sc as plsc
from jax._src.pallas import mpmd as pl_mpmd

_VOCAB = 1024
_SEQ_BLK = 512


def _fill_body(out_ref):
    j = pl.program_id(1)
    seq_len = pl.num_programs(1) * out_ref.shape[1]
    s = out_ref.shape[1]

    col = jax.lax.broadcasted_iota(jnp.int32, (s, _VOCAB), 1)
    pos = j * s + jax.lax.broadcasted_iota(jnp.int32, (s, _VOCAB), 0)

    base = jnp.where(
        col == 0, 1.0, jnp.where(col == 1, 0.5, jnp.where(col == 2, 0.0, -1000.0))
    )
    last_row = jnp.where(col == 0, 0.0, -1000.0)
    out_ref[0] = jnp.where(pos == seq_len - 1, last_row, base)


def _make_scatter(bsz, seq_len):
    n_pos = bsz * seq_len
    info = plsc.get_sparse_core_info()
    n_workers = info.num_cores * info.num_subcores
    per_w = n_pos // n_workers
    n_rows = per_w // 128
    mesh = plsc.VectorSubcoreMesh(core_axis_name="c", subcore_axis_name="s")

    def scatter(tgt_hbm, bg_hbm, out_ref, tgt_v, idx_v, val_v, sem):
        del bg_hbm
        wid = lax.axis_index("s") * info.num_cores + lax.axis_index("c")
        q0 = wid * per_w
        pltpu.sync_copy(tgt_hbm.at[pl.ds(q0, per_w)], tgt_v)
        for r in range(n_rows):
            for c in range(8):
                j = r * 8 + c
                qv = q0 + j * 16 + lax.iota(jnp.int32, 16)
                tgt = tgt_v[pl.ds(j * 16, 16)]
                is_last = (qv & (seq_len - 1)) == (seq_len - 1)
                idx = qv * _VOCAB + jnp.where(is_last, _VOCAB - 1, tgt)
                idx_v[r, pl.ds(c * 16, 16)] = idx
                val_v[r, pl.ds(c * 16, 16)] = jnp.where(is_last, -1000.0, 5.0)
        copies = [
            pltpu.async_copy(val_v.at[r], out_ref.at[idx_v.at[r]], sem)
            for r in range(n_rows)
        ]
        for cp in copies:
            cp.wait()

    return pl_mpmd._mpmd_map(
        [(mesh, scatter)],
        out_types=jax.ShapeDtypeStruct((n_pos * _VOCAB,), jnp.float32),
        input_output_aliases={1: 0},
        scratch_types=[
            pltpu.VMEM((per_w,), jnp.int32),
            pltpu.VMEM((n_rows, 128), jnp.int32),
            pltpu.VMEM((n_rows, 128), jnp.float32),
            pltpu.SemaphoreType.DMA,
        ],
    )


def kernel(input_ids):
    bsz, seq_len = input_ids.shape
    loss = jnp.asarray(0.25, dtype=jnp.float32)
    n_blk = seq_len // _SEQ_BLK

    background = pl.pallas_call(
        _fill_body,
        grid=(bsz, n_blk),
        out_specs=pl.BlockSpec((1, _SEQ_BLK, _VOCAB), lambda b, j: (b, j, 0)),
        out_shape=jax.ShapeDtypeStruct((bsz, seq_len, _VOCAB), jnp.float32),
    )()

    targets = jnp.concatenate(
        [input_ids[:, 1:] % _VOCAB, jnp.zeros((bsz, 1), jnp.int32)], axis=1
    ).reshape(-1)

    flat = _make_scatter(bsz, seq_len)(targets, background.reshape(-1))
    logits = flat.reshape(bsz, seq_len, _VOCAB)
    return loss, logits


# speedup vs baseline: 7.2687x; 7.2468x over previous
import jax
import jax.numpy as jnp
from jax.experimental import pallas as pl

_VOCAB = 1024
_SEQ_BLK = 1024


def _fill_body(tgt_ref, out_ref):
    b = pl.program_id(0)
    j = pl.program_id(1)
    seq_len = tgt_ref.shape[1]
    s = out_ref.shape[1]

    col = jax.lax.broadcasted_iota(jnp.int32, (s, _VOCAB), 1)
    pos = j * s + jax.lax.broadcasted_iota(jnp.int32, (s, _VOCAB), 0)

    p0 = j * s
    tgt = tgt_ref[b, pl.ds(p0, s)]

    base = jnp.where(
        col == 0, 1.0, jnp.where(col == 1, 0.5, jnp.where(col == 2, 0.0, -1000.0))
    )
    vals = jnp.where(col == tgt[:, None], 5.0, base)
    last_row = jnp.where(col == 0, 0.0, -1000.0)
    vals = jnp.where(pos == seq_len - 1, last_row, vals)
    out_ref[0] = vals.astype(jnp.float32)


def kernel(input_ids):
    bsz, seq_len = input_ids.shape
    loss = jnp.asarray(0.25, dtype=jnp.float32)
    n_blk = seq_len // _SEQ_BLK
    targets = jnp.concatenate(
        [input_ids[:, 1:] % _VOCAB, jnp.full((bsz, 1), -1, jnp.int32)], axis=1
    )
    logits = pl.pallas_call(
        _fill_body,
        grid=(bsz, n_blk),
        in_specs=[pl.BlockSpec((bsz, seq_len), lambda b, j: (0, 0))],
        out_specs=pl.BlockSpec((1, _SEQ_BLK, _VOCAB), lambda b, j: (b, j, 0)),
        out_shape=jax.ShapeDtypeStruct((bsz, seq_len, _VOCAB), jnp.float32),
    )(targets)
    return loss, logits
